# baseline (speedup 1.0000x reference)
import functools
import math

import jax
import jax.numpy as jnp
import numpy as np
from jax import lax
from jax.experimental import pallas as pl
from jax.experimental.pallas import tpu as pltpu
from jax.experimental.pallas import tpu_sc as plsc

_ALPHA = 0.1
_THETA = 0.5
_N_LAYERS = 4

_LANES = 16
_NC = 2
_NS = 16
_NW = _NC * _NS
_CHUNK = 128


def _sc_mesh():
    return plsc.VectorSubcoreMesh(core_axis_name="c", subcore_axis_name="s")


def _make_deg_kernel(n_chunks, acc_rows):
    slice_rows = acc_rows // _NS

    @functools.partial(
        pl.kernel,
        out_type=jax.ShapeDtypeStruct((_NC, acc_rows), jnp.float32),
        mesh=_sc_mesh(),
        scratch_types=[
            pltpu.VMEM((n_chunks, _CHUNK), jnp.int32),
            pltpu.VMEM((_CHUNK,), jnp.float32),
            pltpu.VMEM((slice_rows,), jnp.float32),
            pltpu.VMEM_SHARED((acc_rows,), jnp.float32),
            pltpu.SemaphoreType.DMA,
        ],
    )
    def deg_kernel(col2d_hbm, ones_hbm, zeros_hbm, out_hbm, idxc_v, ones_v,
                   zero_v, acc_sh, sem):
        c = lax.axis_index("c")
        s = lax.axis_index("s")
        w = c * _NS + s
        pltpu.sync_copy(zeros_hbm, zero_v)
        pltpu.sync_copy(zero_v, acc_sh.at[pl.ds(s * slice_rows, slice_rows)])
        pltpu.sync_copy(ones_hbm, ones_v)
        pltpu.sync_copy(col2d_hbm.at[pl.ds(w * n_chunks, n_chunks)], idxc_v)
        plsc.subcore_barrier()

        def body(j, carry):
            pltpu.sync_copy(ones_v, acc_sh.at[idxc_v.at[j]], add=True)
            return carry

        lax.fori_loop(0, n_chunks, body, 0)
        plsc.subcore_barrier()
        pltpu.sync_copy(acc_sh.at[pl.ds(s * slice_rows, slice_rows)],
                        out_hbm.at[c, pl.ds(s * slice_rows, slice_rows)])

    return deg_kernel


_GROWS = 64
_RING = 8


def _make_prop_kernel(n_chunks_t, acc_rows, d):
    slice_rows = acc_rows // _NS
    n_quads = n_chunks_t // 4

    @functools.partial(
        pl.kernel,
        out_type=jax.ShapeDtypeStruct((_NC, acc_rows, d), jnp.float32),
        mesh=_sc_mesh(),
        scratch_types=[
            pltpu.VMEM((n_chunks_t // 2, 2 * _GROWS), jnp.int32),
            pltpu.VMEM((_RING, _GROWS), jnp.int32),
            pltpu.VMEM((_RING, _GROWS), jnp.int32),
            [pltpu.VMEM((_GROWS, d), jnp.float32)] * 4,
            pltpu.VMEM_SHARED((acc_rows, d), jnp.float32),
            [pltpu.SemaphoreType.DMA] * 4,
            [pltpu.SemaphoreType.DMA] * 4,
        ],
    )
    def prop_kernel(g_hbm, pack_hbm, zeros_hbm, out_hbm,
                    idxp_v, ring_r, ring_c, bufs, acc_sh, semg, sems):
        c = lax.axis_index("c")
        s = lax.axis_index("s")
        w = c * _NS + s
        n_rows_p = n_chunks_t // 2

        def unpack_chunks(j):
            rp = j // 2
            for half in range(2):
                slot = lax.rem(j + half, _RING)
                for q in range(_GROWS // _LANES):
                    v = idxp_v[rp, pl.ds(half * _GROWS + q * _LANES, _LANES)]
                    ring_r[slot, pl.ds(q * _LANES, _LANES)] = v & 0xFFFF
                    ring_c[slot, pl.ds(q * _LANES, _LANES)] = (
                        lax.shift_right_logical(v, 16))

        pltpu.sync_copy(zeros_hbm, bufs[0])
        for z in range(slice_rows // _GROWS):
            pltpu.sync_copy(
                bufs[0], acc_sh.at[pl.ds(s * slice_rows + z * _GROWS, _GROWS)])
        pltpu.sync_copy(pack_hbm.at[pl.ds(w * n_rows_p, n_rows_p)], idxp_v)
        unpack_chunks(0)
        unpack_chunks(2)
        for b in range(4):
            pltpu.async_copy(g_hbm.at[ring_r.at[b]], bufs[b], semg[b])
        plsc.subcore_barrier()

        def quad(jj, carry):
            j = 4 * jj
            @pl.when(jj < n_quads - 1)
            def _():
                unpack_chunks(j + 4)
                unpack_chunks(j + 6)

            for b in range(4):
                slot = lax.rem(j + b, _RING)
                pltpu.make_async_copy(
                    g_hbm.at[ring_r.at[slot]], bufs[b], semg[b]).wait()
                pltpu.async_copy(
                    bufs[b], acc_sh.at[ring_c.at[slot]], sems[b], add=True)
            for b in range(4):
                slot = lax.rem(j + b, _RING)
                slot4 = lax.rem(j + b + 4, _RING)
                pltpu.make_async_copy(
                    bufs[b], acc_sh.at[ring_c.at[slot]], sems[b]).wait()

                @pl.when(jj < n_quads - 1)
                def _():
                    pltpu.async_copy(
                        g_hbm.at[ring_r.at[slot4]], bufs[b], semg[b])

            return carry

        lax.fori_loop(0, n_quads, quad, 0)
        plsc.subcore_barrier()
        pltpu.sync_copy(acc_sh.at[pl.ds(s * slice_rows, slice_rows)],
                        out_hbm.at[c, pl.ds(s * slice_rows, slice_rows)])

    return prop_kernel


_BLK = 1000


def _tc_input_kernel(x_ref, w_ref, b_ref, d0_ref, d1_ref, dinv_ref, g_ref):
    h = jnp.dot(x_ref[...], w_ref[...], preferred_element_type=jnp.float32)
    h = jnp.maximum(h + b_ref[...], 0.0)
    dinv = lax.rsqrt(d0_ref[...] + d1_ref[...] + 1.0)
    dinv_ref[...] = dinv
    g_ref[...] = dinv * h


def _tc_layer0_kernel(s_ref, g_ref, dinv_ref, x0_ref, gout_ref):
    dinv = dinv_ref[...]
    agg = dinv * (s_ref[0] + s_ref[1] + g_ref[...])
    x0 = jnp.maximum(agg, 0.0)
    x0_ref[...] = x0
    gout_ref[...] = dinv * x0


def _tc_mid_kernel(beta, s_ref, g_ref, x0_ref, dinv_ref, w_ref, gout_ref):
    dinv = dinv_ref[...]
    agg = dinv * (s_ref[0] + s_ref[1] + g_ref[...])
    xp = agg * (1.0 - _ALPHA) + _ALPHA * x0_ref[...]
    out = (1.0 - beta) * xp + beta * jnp.dot(
        xp, w_ref[...], preferred_element_type=jnp.float32)
    gout_ref[...] = dinv * jnp.maximum(out, 0.0)


def _tc_final_kernel(beta, s_ref, g_ref, x0_ref, dinv_ref, w_ref,
                     fw_ref, fb_ref, out_ref):
    dinv = dinv_ref[...]
    agg = dinv * (s_ref[0] + s_ref[1] + g_ref[...])
    xp = agg * (1.0 - _ALPHA) + _ALPHA * x0_ref[...]
    out = (1.0 - beta) * xp + beta * jnp.dot(
        xp, w_ref[...], preferred_element_type=jnp.float32)
    h = jnp.maximum(out, 0.0)
    out_ref[...] = jnp.dot(h, fw_ref[...],
                           preferred_element_type=jnp.float32) + fb_ref[...]


def _node_spec(d):
    return pl.BlockSpec((_BLK, d), lambda i: (i, 0))


def _s_spec(d):
    return pl.BlockSpec((2, _BLK, d), lambda i: (0, i, 0))


def _full_spec(r, c):
    return pl.BlockSpec((r, c), lambda i: (0, 0))


def kernel(x, edge_index, fc0_w, fc0_b, fc1_w, fc1_b, conv_w):
    n, d = x.shape
    e = edge_index.shape[1]
    n_classes = fc1_w.shape[1]

    grain = _NW * _CHUNK
    n_chunks = pl.cdiv(pl.cdiv(e, grain), 8) * 8
    e_pad = n_chunks * grain
    acc_rows = pl.cdiv(n + 64, 640) * 640
    slice_rows = acc_rows // _NS
    assert slice_rows % _CHUNK == 0

    pad = e_pad - e
    pad_ar = np.arange(pad, dtype=np.int32)
    pad_pack = jnp.asarray(((n + pad_ar % 128) << 16) | (pad_ar % 128))
    wvec = jnp.array([[1], [1 << 16]], dtype=jnp.int32)
    packed_e = jnp.sum(edge_index * wvec, axis=0, dtype=jnp.int32)
    pack2d = jnp.concatenate([packed_e, pad_pack]).reshape(-1, 2 * _GROWS)
    col2d = pack2d >> 16
    n_chunks_t = e_pad // (_NW * _GROWS)

    ones1 = jnp.ones((_CHUNK,), jnp.float32)
    zeros1 = jnp.zeros((slice_rows,), jnp.float32)
    zeros2 = jnp.zeros((_GROWS, d), jnp.float32)

    deg_k = _make_deg_kernel(n_chunks, acc_rows)
    prop_k = _make_prop_kernel(n_chunks_t, acc_rows, d)

    deg = deg_k(col2d, ones1, zeros1)
    deg0 = deg[0, :n].reshape(n, 1)
    deg1 = deg[1, :n].reshape(n, 1)

    grid = n // _BLK
    dinv, g = pl.pallas_call(
        _tc_input_kernel,
        grid=(grid,),
        in_specs=[_node_spec(d), _full_spec(d, d), _full_spec(1, d),
                  _node_spec(1), _node_spec(1)],
        out_specs=[_node_spec(1), _node_spec(d)],
        out_shape=[jax.ShapeDtypeStruct((n, 1), jnp.float32),
                   jax.ShapeDtypeStruct((n, d), jnp.float32)],
    )(x, fc0_w, fc0_b.reshape(1, d), deg0, deg1)

    s = prop_k(g, pack2d, zeros2)
    x0, g = pl.pallas_call(
        _tc_layer0_kernel,
        grid=(grid,),
        in_specs=[_s_spec(d), _node_spec(d), _node_spec(1)],
        out_specs=[_node_spec(d), _node_spec(d)],
        out_shape=[jax.ShapeDtypeStruct((n, d), jnp.float32),
                   jax.ShapeDtypeStruct((n, d), jnp.float32)],
    )(s, g, dinv)

    for i in range(1, _N_LAYERS - 1):
        beta = math.log(_THETA / (i + 1) + 1.0)
        s = prop_k(g, pack2d, zeros2)
        g = pl.pallas_call(
            functools.partial(_tc_mid_kernel, beta),
            grid=(grid,),
            in_specs=[_s_spec(d), _node_spec(d), _node_spec(d),
                      _node_spec(1), _full_spec(d, d)],
            out_specs=_node_spec(d),
            out_shape=jax.ShapeDtypeStruct((n, d), jnp.float32),
        )(s, g, x0, dinv, conv_w[i])

    i = _N_LAYERS - 1
    beta = math.log(_THETA / (i + 1) + 1.0)
    s = prop_k(g, pack2d, zeros2)
    out = pl.pallas_call(
        functools.partial(_tc_final_kernel, beta),
        grid=(grid,),
        in_specs=[_s_spec(d), _node_spec(d), _node_spec(d),
                  _node_spec(1), _full_spec(d, d), _full_spec(d, n_classes),
                  _full_spec(1, n_classes)],
        out_specs=_node_spec(n_classes),
        out_shape=jax.ShapeDtypeStruct((n, n_classes), jnp.float32),
    )(s, g, x0, dinv, conv_w[i], fc1_w, fc1_b.reshape(1, n_classes))
    return out


# speedup vs baseline: 1.0326x; 1.0326x over previous
import functools
import math

import jax
import jax.numpy as jnp
import numpy as np
from jax import lax
from jax.experimental import pallas as pl
from jax.experimental.pallas import tpu as pltpu
from jax.experimental.pallas import tpu_sc as plsc

_ALPHA = 0.1
_THETA = 0.5
_N_LAYERS = 4

_LANES = 16
_NC = 2
_NS = 16
_NW = _NC * _NS
_CHUNK = 128


def _sc_mesh():
    return plsc.VectorSubcoreMesh(core_axis_name="c", subcore_axis_name="s")


def _make_deg_kernel(n_chunks, acc_rows):
    slice_rows = acc_rows // _NS

    @functools.partial(
        pl.kernel,
        out_type=jax.ShapeDtypeStruct((_NC, acc_rows), jnp.float32),
        mesh=_sc_mesh(),
        scratch_types=[
            pltpu.VMEM((n_chunks, _CHUNK), jnp.int32),
            pltpu.VMEM((_CHUNK,), jnp.float32),
            pltpu.VMEM((slice_rows,), jnp.float32),
            pltpu.VMEM_SHARED((acc_rows,), jnp.float32),
            pltpu.SemaphoreType.DMA,
        ],
    )
    def deg_kernel(col2d_hbm, ones_hbm, zeros_hbm, out_hbm, idxc_v, ones_v,
                   zero_v, acc_sh, sem):
        c = lax.axis_index("c")
        s = lax.axis_index("s")
        w = c * _NS + s
        pltpu.sync_copy(zeros_hbm, zero_v)
        pltpu.sync_copy(zero_v, acc_sh.at[pl.ds(s * slice_rows, slice_rows)])
        pltpu.sync_copy(ones_hbm, ones_v)
        pltpu.sync_copy(col2d_hbm.at[pl.ds(w * n_chunks, n_chunks)], idxc_v)
        plsc.subcore_barrier()

        def body(j, carry):
            pltpu.async_copy(ones_v, acc_sh.at[idxc_v.at[j]], sem, add=True)
            return carry

        lax.fori_loop(0, n_chunks, body, 0)

        def drain(j, carry):
            pltpu.make_async_copy(ones_v, acc_sh.at[idxc_v.at[j]], sem).wait()
            return carry

        lax.fori_loop(0, n_chunks, drain, 0)
        plsc.subcore_barrier()
        pltpu.sync_copy(acc_sh.at[pl.ds(s * slice_rows, slice_rows)],
                        out_hbm.at[c, pl.ds(s * slice_rows, slice_rows)])

    return deg_kernel


_GROWS = 64
_RING = 8


def _make_prop_kernel(n_chunks_t, acc_rows, d):
    slice_rows = acc_rows // _NS
    n_quads = n_chunks_t // 4

    @functools.partial(
        pl.kernel,
        out_type=jax.ShapeDtypeStruct((_NC, acc_rows, d), jnp.float32),
        mesh=_sc_mesh(),
        scratch_types=[
            pltpu.VMEM((n_chunks_t // 2, 2 * _GROWS), jnp.int32),
            pltpu.VMEM((_RING, _GROWS), jnp.int32),
            pltpu.VMEM((_RING, _GROWS), jnp.int32),
            [pltpu.VMEM((_GROWS, d), jnp.float32)] * 4,
            pltpu.VMEM_SHARED((acc_rows, d), jnp.float32),
            [pltpu.SemaphoreType.DMA] * 4,
            [pltpu.SemaphoreType.DMA] * 4,
        ],
    )
    def prop_kernel(g_hbm, pack_hbm, zeros_hbm, out_hbm,
                    idxp_v, ring_r, ring_c, bufs, acc_sh, semg, sems):
        c = lax.axis_index("c")
        s = lax.axis_index("s")
        w = c * _NS + s
        n_rows_p = n_chunks_t // 2

        def unpack_chunks(j):
            rp = j // 2
            for half in range(2):
                slot = lax.rem(j + half, _RING)
                for q in range(_GROWS // _LANES):
                    v = idxp_v[rp, pl.ds(half * _GROWS + q * _LANES, _LANES)]
                    ring_r[slot, pl.ds(q * _LANES, _LANES)] = v & 0xFFFF
                    ring_c[slot, pl.ds(q * _LANES, _LANES)] = (
                        lax.shift_right_logical(v, 16))

        pltpu.sync_copy(zeros_hbm, bufs[0])
        for z in range(slice_rows // _GROWS):
            pltpu.sync_copy(
                bufs[0], acc_sh.at[pl.ds(s * slice_rows + z * _GROWS, _GROWS)])
        pltpu.sync_copy(pack_hbm.at[pl.ds(w * n_rows_p, n_rows_p)], idxp_v)
        unpack_chunks(0)
        unpack_chunks(2)
        for b in range(4):
            pltpu.async_copy(g_hbm.at[ring_r.at[b]], bufs[b], semg[b])
        plsc.subcore_barrier()

        def quad(jj, carry):
            j = 4 * jj
            @pl.when(jj < n_quads - 1)
            def _():
                unpack_chunks(j + 4)
                unpack_chunks(j + 6)

            for b in range(4):
                slot = lax.rem(j + b, _RING)
                pltpu.make_async_copy(
                    g_hbm.at[ring_r.at[slot]], bufs[b], semg[b]).wait()
                pltpu.async_copy(
                    bufs[b], acc_sh.at[ring_c.at[slot]], sems[b], add=True)
            for b in range(4):
                slot = lax.rem(j + b, _RING)
                slot4 = lax.rem(j + b + 4, _RING)
                pltpu.make_async_copy(
                    bufs[b], acc_sh.at[ring_c.at[slot]], sems[b]).wait()

                @pl.when(jj < n_quads - 1)
                def _():
                    pltpu.async_copy(
                        g_hbm.at[ring_r.at[slot4]], bufs[b], semg[b])

            return carry

        lax.fori_loop(0, n_quads, quad, 0)
        plsc.subcore_barrier()
        pltpu.sync_copy(acc_sh.at[pl.ds(s * slice_rows, slice_rows)],
                        out_hbm.at[c, pl.ds(s * slice_rows, slice_rows)])

    return prop_kernel


_BLK = 2000


def _tc_input_kernel(x_ref, w_ref, b_ref, d0_ref, d1_ref, dinv_ref, g_ref):
    h = jnp.dot(x_ref[...], w_ref[...], preferred_element_type=jnp.float32)
    h = jnp.maximum(h + b_ref[...], 0.0)
    dinv = lax.rsqrt(d0_ref[...] + d1_ref[...] + 1.0)
    dinv_ref[...] = dinv
    g_ref[...] = dinv * h


def _tc_layer0_kernel(s_ref, g_ref, dinv_ref, x0_ref, gout_ref):
    dinv = dinv_ref[...]
    agg = dinv * (s_ref[0] + s_ref[1] + g_ref[...])
    x0 = jnp.maximum(agg, 0.0)
    x0_ref[...] = x0
    gout_ref[...] = dinv * x0


def _tc_mid_kernel(beta, s_ref, g_ref, x0_ref, dinv_ref, w_ref, gout_ref):
    dinv = dinv_ref[...]
    agg = dinv * (s_ref[0] + s_ref[1] + g_ref[...])
    xp = agg * (1.0 - _ALPHA) + _ALPHA * x0_ref[...]
    out = (1.0 - beta) * xp + beta * jnp.dot(
        xp, w_ref[...], preferred_element_type=jnp.float32)
    gout_ref[...] = dinv * jnp.maximum(out, 0.0)


def _tc_final_kernel(beta, s_ref, g_ref, x0_ref, dinv_ref, w_ref,
                     fw_ref, fb_ref, out_ref):
    dinv = dinv_ref[...]
    agg = dinv * (s_ref[0] + s_ref[1] + g_ref[...])
    xp = agg * (1.0 - _ALPHA) + _ALPHA * x0_ref[...]
    out = (1.0 - beta) * xp + beta * jnp.dot(
        xp, w_ref[...], preferred_element_type=jnp.float32)
    h = jnp.maximum(out, 0.0)
    out_ref[...] = jnp.dot(h, fw_ref[...],
                           preferred_element_type=jnp.float32) + fb_ref[...]


def _node_spec(d):
    return pl.BlockSpec((_BLK, d), lambda i: (i, 0))


def _s_spec(d):
    return pl.BlockSpec((2, _BLK, d), lambda i: (0, i, 0))


def _full_spec(r, c):
    return pl.BlockSpec((r, c), lambda i: (0, 0))


def kernel(x, edge_index, fc0_w, fc0_b, fc1_w, fc1_b, conv_w):
    n, d = x.shape
    e = edge_index.shape[1]
    n_classes = fc1_w.shape[1]

    grain = _NW * _CHUNK
    n_chunks = pl.cdiv(pl.cdiv(e, grain), 8) * 8
    e_pad = n_chunks * grain
    acc_rows = pl.cdiv(n + 64, 640) * 640
    slice_rows = acc_rows // _NS
    assert slice_rows % _CHUNK == 0

    pad = e_pad - e
    pad_ar = np.arange(pad, dtype=np.int32)
    pad_row = jnp.asarray(pad_ar % 128)
    pad_col = jnp.asarray(n + pad_ar % 128)
    row_p = jnp.concatenate([edge_index[0], pad_row])
    col_p = jnp.concatenate([edge_index[1], pad_col])
    col2d = col_p.reshape(-1, _CHUNK)
    pack2d = ((col_p << 16) | row_p).reshape(-1, 2 * _GROWS)
    n_chunks_t = e_pad // (_NW * _GROWS)

    ones1 = jnp.ones((_CHUNK,), jnp.float32)
    zeros1 = jnp.zeros((slice_rows,), jnp.float32)
    zeros2 = jnp.zeros((_GROWS, d), jnp.float32)

    deg_k = _make_deg_kernel(n_chunks, acc_rows)
    prop_k = _make_prop_kernel(n_chunks_t, acc_rows, d)

    deg = deg_k(col2d, ones1, zeros1)
    deg0 = deg[0, :n].reshape(n, 1)
    deg1 = deg[1, :n].reshape(n, 1)

    grid = n // _BLK
    dinv, g = pl.pallas_call(
        _tc_input_kernel,
        grid=(grid,),
        in_specs=[_node_spec(d), _full_spec(d, d), _full_spec(1, d),
                  _node_spec(1), _node_spec(1)],
        out_specs=[_node_spec(1), _node_spec(d)],
        out_shape=[jax.ShapeDtypeStruct((n, 1), jnp.float32),
                   jax.ShapeDtypeStruct((n, d), jnp.float32)],
    )(x, fc0_w, fc0_b.reshape(1, d), deg0, deg1)

    s = prop_k(g, pack2d, zeros2)
    x0, g = pl.pallas_call(
        _tc_layer0_kernel,
        grid=(grid,),
        in_specs=[_s_spec(d), _node_spec(d), _node_spec(1)],
        out_specs=[_node_spec(d), _node_spec(d)],
        out_shape=[jax.ShapeDtypeStruct((n, d), jnp.float32),
                   jax.ShapeDtypeStruct((n, d), jnp.float32)],
    )(s, g, dinv)

    for i in range(1, _N_LAYERS - 1):
        beta = math.log(_THETA / (i + 1) + 1.0)
        s = prop_k(g, pack2d, zeros2)
        g = pl.pallas_call(
            functools.partial(_tc_mid_kernel, beta),
            grid=(grid,),
            in_specs=[_s_spec(d), _node_spec(d), _node_spec(d),
                      _node_spec(1), _full_spec(d, d)],
            out_specs=_node_spec(d),
            out_shape=jax.ShapeDtypeStruct((n, d), jnp.float32),
        )(s, g, x0, dinv, conv_w[i])

    i = _N_LAYERS - 1
    beta = math.log(_THETA / (i + 1) + 1.0)
    s = prop_k(g, pack2d, zeros2)
    out = pl.pallas_call(
        functools.partial(_tc_final_kernel, beta),
        grid=(grid,),
        in_specs=[_s_spec(d), _node_spec(d), _node_spec(d),
                  _node_spec(1), _full_spec(d, d), _full_spec(d, n_classes),
                  _full_spec(1, n_classes)],
        out_specs=_node_spec(n_classes),
        out_shape=jax.ShapeDtypeStruct((n, n_classes), jnp.float32),
    )(s, g, x0, dinv, conv_w[i], fc1_w, fc1_b.reshape(1, n_classes))
    return out


# speedup vs baseline: 1.0490x; 1.0159x over previous
import functools
import math

import jax
import jax.numpy as jnp
import numpy as np
from jax import lax
from jax.experimental import pallas as pl
from jax.experimental.pallas import tpu as pltpu
from jax.experimental.pallas import tpu_sc as plsc

_ALPHA = 0.1
_THETA = 0.5
_N_LAYERS = 4

_LANES = 16
_NC = 2
_NS = 16
_NW = _NC * _NS
_CHUNK = 128


def _sc_mesh():
    return plsc.VectorSubcoreMesh(core_axis_name="c", subcore_axis_name="s")


def _make_deg_kernel(n_chunks, acc_rows):
    slice_rows = acc_rows // _NS

    @functools.partial(
        pl.kernel,
        out_type=jax.ShapeDtypeStruct((_NC, acc_rows), jnp.float32),
        mesh=_sc_mesh(),
        scratch_types=[
            pltpu.VMEM((n_chunks, _CHUNK), jnp.int32),
            pltpu.VMEM((_CHUNK,), jnp.float32),
            pltpu.VMEM((slice_rows,), jnp.float32),
            pltpu.VMEM_SHARED((acc_rows,), jnp.float32),
            pltpu.SemaphoreType.DMA,
        ],
    )
    def deg_kernel(col2d_hbm, ones_hbm, zeros_hbm, out_hbm, idxc_v, ones_v,
                   zero_v, acc_sh, sem):
        c = lax.axis_index("c")
        s = lax.axis_index("s")
        w = c * _NS + s
        pltpu.sync_copy(zeros_hbm, zero_v)
        pltpu.sync_copy(zero_v, acc_sh.at[pl.ds(s * slice_rows, slice_rows)])
        pltpu.sync_copy(ones_hbm, ones_v)
        pltpu.sync_copy(col2d_hbm.at[pl.ds(w * n_chunks, n_chunks)], idxc_v)
        plsc.subcore_barrier()

        def body(j, carry):
            pltpu.async_copy(ones_v, acc_sh.at[idxc_v.at[j]], sem, add=True)
            return carry

        lax.fori_loop(0, n_chunks, body, 0)

        def drain(j, carry):
            pltpu.make_async_copy(ones_v, acc_sh.at[idxc_v.at[j]], sem).wait()
            return carry

        lax.fori_loop(0, n_chunks, drain, 0)
        plsc.subcore_barrier()
        pltpu.sync_copy(acc_sh.at[pl.ds(s * slice_rows, slice_rows)],
                        out_hbm.at[c, pl.ds(s * slice_rows, slice_rows)])

    return deg_kernel


_GROWS = 64
_RING = 8


def _make_prop_kernel(n_chunks_t, acc_rows, d):
    slice_rows = acc_rows // _NS
    n_quads = n_chunks_t // 4

    @functools.partial(
        pl.kernel,
        out_type=jax.ShapeDtypeStruct((_NC, acc_rows, d), jnp.float32),
        mesh=_sc_mesh(),
        scratch_types=[
            pltpu.VMEM((n_chunks_t // 2, 2 * _GROWS), jnp.int32),
            pltpu.VMEM((_RING, _GROWS), jnp.int32),
            pltpu.VMEM((_RING, _GROWS), jnp.int32),
            [pltpu.VMEM((_GROWS, d), jnp.float32)] * 4,
            pltpu.VMEM_SHARED((acc_rows, d), jnp.float32),
            [pltpu.SemaphoreType.DMA] * 4,
            [pltpu.SemaphoreType.DMA] * 4,
        ],
    )
    def prop_kernel(g_hbm, pack_hbm, zeros_hbm, out_hbm,
                    idxp_v, ring_r, ring_c, bufs, acc_sh, semg, sems):
        c = lax.axis_index("c")
        s = lax.axis_index("s")
        w = c * _NS + s
        n_rows_p = n_chunks_t // 2

        def unpack_chunks(j):
            rp = j // 2
            for half in range(2):
                slot = lax.rem(j + half, _RING)
                for q in range(_GROWS // _LANES):
                    v = idxp_v[rp, pl.ds(half * _GROWS + q * _LANES, _LANES)]
                    ring_r[slot, pl.ds(q * _LANES, _LANES)] = v & 0xFFFF
                    ring_c[slot, pl.ds(q * _LANES, _LANES)] = (
                        lax.shift_right_logical(v, 16))

        pltpu.sync_copy(zeros_hbm, bufs[0])
        for z in range(slice_rows // _GROWS):
            pltpu.async_copy(
                bufs[0], acc_sh.at[pl.ds(s * slice_rows + z * _GROWS, _GROWS)],
                sems[0])
        pltpu.sync_copy(pack_hbm.at[pl.ds(w * n_rows_p, n_rows_p)], idxp_v)
        unpack_chunks(0)
        unpack_chunks(2)
        for b in range(1, 4):
            pltpu.async_copy(g_hbm.at[ring_r.at[b]], bufs[b], semg[b])
        for z in range(slice_rows // _GROWS):
            pltpu.make_async_copy(
                bufs[0], acc_sh.at[pl.ds(s * slice_rows + z * _GROWS, _GROWS)],
                sems[0]).wait()
        pltpu.async_copy(g_hbm.at[ring_r.at[0]], bufs[0], semg[0])
        plsc.subcore_barrier()

        def quad(jj, carry):
            j = 4 * jj
            @pl.when(jj < n_quads - 1)
            def _():
                unpack_chunks(j + 4)
                unpack_chunks(j + 6)

            for b in range(4):
                slot = lax.rem(j + b, _RING)
                pltpu.make_async_copy(
                    g_hbm.at[ring_r.at[slot]], bufs[b], semg[b]).wait()
                pltpu.async_copy(
                    bufs[b], acc_sh.at[ring_c.at[slot]], sems[b], add=True)
            for b in range(4):
                slot = lax.rem(j + b, _RING)
                slot4 = lax.rem(j + b + 4, _RING)
                pltpu.make_async_copy(
                    bufs[b], acc_sh.at[ring_c.at[slot]], sems[b]).wait()

                @pl.when(jj < n_quads - 1)
                def _():
                    pltpu.async_copy(
                        g_hbm.at[ring_r.at[slot4]], bufs[b], semg[b])

            return carry

        lax.fori_loop(0, n_quads, quad, 0)
        plsc.subcore_barrier()
        pltpu.sync_copy(acc_sh.at[pl.ds(s * slice_rows, slice_rows)],
                        out_hbm.at[c, pl.ds(s * slice_rows, slice_rows)])

    return prop_kernel


_BLK = 2000


def _tc_input_kernel(x_ref, w_ref, b_ref, d0_ref, d1_ref, dinv_ref, g_ref):
    h = jnp.dot(x_ref[...], w_ref[...], preferred_element_type=jnp.float32)
    h = jnp.maximum(h + b_ref[...], 0.0)
    dinv = lax.rsqrt(d0_ref[...] + d1_ref[...] + 1.0)
    dinv_ref[...] = dinv
    g_ref[...] = dinv * h


def _tc_layer0_kernel(s_ref, g_ref, dinv_ref, x0_ref, gout_ref):
    dinv = dinv_ref[...]
    agg = dinv * (s_ref[0] + s_ref[1] + g_ref[...])
    x0 = jnp.maximum(agg, 0.0)
    x0_ref[...] = x0
    gout_ref[...] = dinv * x0


def _tc_mid_kernel(beta, s_ref, g_ref, x0_ref, dinv_ref, w_ref, gout_ref):
    dinv = dinv_ref[...]
    agg = dinv * (s_ref[0] + s_ref[1] + g_ref[...])
    xp = agg * (1.0 - _ALPHA) + _ALPHA * x0_ref[...]
    out = (1.0 - beta) * xp + beta * jnp.dot(
        xp, w_ref[...], preferred_element_type=jnp.float32)
    gout_ref[...] = dinv * jnp.maximum(out, 0.0)


def _tc_final_kernel(beta, s_ref, g_ref, x0_ref, dinv_ref, w_ref,
                     fw_ref, fb_ref, out_ref):
    dinv = dinv_ref[...]
    agg = dinv * (s_ref[0] + s_ref[1] + g_ref[...])
    xp = agg * (1.0 - _ALPHA) + _ALPHA * x0_ref[...]
    out = (1.0 - beta) * xp + beta * jnp.dot(
        xp, w_ref[...], preferred_element_type=jnp.float32)
    h = jnp.maximum(out, 0.0)
    out_ref[...] = jnp.dot(h, fw_ref[...],
                           preferred_element_type=jnp.float32) + fb_ref[...]


def _node_spec(d):
    return pl.BlockSpec((_BLK, d), lambda i: (i, 0))


def _s_spec(d):
    return pl.BlockSpec((2, _BLK, d), lambda i: (0, i, 0))


def _full_spec(r, c):
    return pl.BlockSpec((r, c), lambda i: (0, 0))


def kernel(x, edge_index, fc0_w, fc0_b, fc1_w, fc1_b, conv_w):
    n, d = x.shape
    e = edge_index.shape[1]
    n_classes = fc1_w.shape[1]

    grain = _NW * _CHUNK
    n_chunks = pl.cdiv(pl.cdiv(e, grain), 8) * 8
    e_pad = n_chunks * grain
    acc_rows = pl.cdiv(n + 64, 640) * 640
    slice_rows = acc_rows // _NS
    assert slice_rows % _CHUNK == 0

    pad = e_pad - e
    pad_ar = np.arange(pad, dtype=np.int32)
    pad_row = jnp.asarray(pad_ar % 128)
    pad_col = jnp.asarray(n + pad_ar % 128)
    row_p = jnp.concatenate([edge_index[0], pad_row])
    col_p = jnp.concatenate([edge_index[1], pad_col])
    col2d = col_p.reshape(-1, _CHUNK)
    pack2d = ((col_p << 16) | row_p).reshape(-1, 2 * _GROWS)
    n_chunks_t = e_pad // (_NW * _GROWS)

    ones1 = jnp.ones((_CHUNK,), jnp.float32)
    zeros1 = jnp.zeros((slice_rows,), jnp.float32)
    zeros2 = jnp.zeros((_GROWS, d), jnp.float32)

    deg_k = _make_deg_kernel(n_chunks, acc_rows)
    prop_k = _make_prop_kernel(n_chunks_t, acc_rows, d)

    deg = deg_k(col2d, ones1, zeros1)
    deg0 = deg[0, :n].reshape(n, 1)
    deg1 = deg[1, :n].reshape(n, 1)

    grid = n // _BLK
    dinv, g = pl.pallas_call(
        _tc_input_kernel,
        grid=(grid,),
        in_specs=[_node_spec(d), _full_spec(d, d), _full_spec(1, d),
                  _node_spec(1), _node_spec(1)],
        out_specs=[_node_spec(1), _node_spec(d)],
        out_shape=[jax.ShapeDtypeStruct((n, 1), jnp.float32),
                   jax.ShapeDtypeStruct((n, d), jnp.float32)],
    )(x, fc0_w, fc0_b.reshape(1, d), deg0, deg1)

    s = prop_k(g, pack2d, zeros2)
    x0, g = pl.pallas_call(
        _tc_layer0_kernel,
        grid=(grid,),
        in_specs=[_s_spec(d), _node_spec(d), _node_spec(1)],
        out_specs=[_node_spec(d), _node_spec(d)],
        out_shape=[jax.ShapeDtypeStruct((n, d), jnp.float32),
                   jax.ShapeDtypeStruct((n, d), jnp.float32)],
    )(s, g, dinv)

    for i in range(1, _N_LAYERS - 1):
        beta = math.log(_THETA / (i + 1) + 1.0)
        s = prop_k(g, pack2d, zeros2)
        g = pl.pallas_call(
            functools.partial(_tc_mid_kernel, beta),
            grid=(grid,),
            in_specs=[_s_spec(d), _node_spec(d), _node_spec(d),
                      _node_spec(1), _full_spec(d, d)],
            out_specs=_node_spec(d),
            out_shape=jax.ShapeDtypeStruct((n, d), jnp.float32),
        )(s, g, x0, dinv, conv_w[i])

    i = _N_LAYERS - 1
    beta = math.log(_THETA / (i + 1) + 1.0)
    s = prop_k(g, pack2d, zeros2)
    out = pl.pallas_call(
        functools.partial(_tc_final_kernel, beta),
        grid=(grid,),
        in_specs=[_s_spec(d), _node_spec(d), _node_spec(d),
                  _node_spec(1), _full_spec(d, d), _full_spec(d, n_classes),
                  _full_spec(1, n_classes)],
        out_specs=_node_spec(n_classes),
        out_shape=jax.ShapeDtypeStruct((n, n_classes), jnp.float32),
    )(s, g, x0, dinv, conv_w[i], fc1_w, fc1_b.reshape(1, n_classes))
    return out
